# baseline (speedup 1.0000x reference)
import jax
import jax.numpy as jnp
from jax.experimental import pallas as pl

_B = 8
_C_IN = 768
_C_OUT = 64
_STRIDE = 12
_HW = 224 * 224


def _copy_kernel(x_ref, o_ref):
    o_ref[...] = x_ref[...]


def kernel(x):
    out = pl.pallas_call(
        _copy_kernel,
        grid=(_B, _C_OUT),
        in_specs=[
            pl.BlockSpec((1, 1, 224, 224), lambda b, c: (b, c * _STRIDE, 0, 0)),
        ],
        out_specs=pl.BlockSpec((1, 1, 224, 224), lambda b, c: (b, c, 0, 0)),
        out_shape=jax.ShapeDtypeStruct((_B, _C_OUT, 224, 224), x.dtype),
    )(x)
    return out
